# baseline (speedup 1.0000x reference)
import functools

import jax
import jax.numpy as jnp
from jax import lax
from jax.experimental import pallas as pl
from jax.experimental.pallas import tpu as pltpu
from jax.experimental.pallas import tpu_sc as plsc

B, N, MAX_NB = 16, 256, 10
ATOM_FDIM, BOND_FDIM, HIDDEN = 82, 6, 300
BN = B * N
D = 304
DT = BN + 8
AF_P = 88
BF_P = 8

NC, NS, L = 2, 16, 16
NW = NC * NS
NPW = BN // NW
CH = 8
ROWS = CH * MAX_NB
NCHUNK = NPW // CH
NCB = D // L


def _tc_prep(atom_ref, bond_ref, wa_ref, wu2a_ref, wnb_ref, wu2b_ref, bu2_ref,
             af_ref, q_ref, hb_ref, fb_ref):
    af = jnp.dot(atom_ref[...], wa_ref[...], preferred_element_type=jnp.float32)
    af_ref[...] = af
    q_ref[:BN, :] = jnp.dot(af, wu2a_ref[...], preferred_element_type=jnp.float32)
    q_ref[BN:, :] = jnp.zeros((DT - BN, D), jnp.float32)
    bond = bond_ref[...]
    hb_ref[:BN, :] = jnp.dot(bond, wnb_ref[...], preferred_element_type=jnp.float32)
    hb_ref[BN:, :] = jnp.zeros((DT - BN, D), jnp.float32)
    fb_ref[:BN, :] = jnp.dot(bond, wu2b_ref[...], preferred_element_type=jnp.float32) + bu2_ref[...]
    fb_ref[BN:, :] = jnp.zeros((DT - BN, D), jnp.float32)


def _tc_mid(af_ref, nl_ref, wu1a_ref, wu1b_ref, bu1_ref, wu2a_ref,
            afn_ref, q_ref):
    h = (jnp.dot(af_ref[...], wu1a_ref[...], preferred_element_type=jnp.float32)
         + jnp.dot(nl_ref[...], wu1b_ref[...], preferred_element_type=jnp.float32)
         + bu1_ref[...])
    afn = jnp.maximum(h, 0.0)
    afn_ref[...] = afn
    q_ref[:BN, :] = jnp.dot(afn, wu2a_ref[...], preferred_element_type=jnp.float32)
    q_ref[BN:, :] = jnp.zeros((DT - BN, D), jnp.float32)


def _tc_last(af_ref, nl_ref, wu1a_ref, wu1b_ref, bu1_ref, wna_ref, ws_ref,
             p_ref, s_ref):
    h = (jnp.dot(af_ref[...], wu1a_ref[...], preferred_element_type=jnp.float32)
         + jnp.dot(nl_ref[...], wu1b_ref[...], preferred_element_type=jnp.float32)
         + bu1_ref[...])
    afn = jnp.maximum(h, 0.0)
    p_ref[:BN, :] = jnp.dot(afn, wna_ref[...], preferred_element_type=jnp.float32)
    p_ref[BN:, :] = jnp.zeros((DT - BN, D), jnp.float32)
    s_ref[...] = jnp.dot(afn, ws_ref[...], preferred_element_type=jnp.float32)


def _tc_out(s_ref, fn_ref, nm_ref, o_ref):
    o_ref[...] = s_ref[...] * fn_ref[...] * nm_ref[...]


def _run_tc(body, out_shapes, *args):
    return pl.pallas_call(
        body,
        out_shape=[jax.ShapeDtypeStruct(s, jnp.float32) for s in out_shapes],
    )(*args)


def _sc_stage_body(mode, t1_hbm, t2_hbm, ia_hbm, ib_hbm, o_hbm,
                   ia_v, ib_v, r1_v, r2_v, o_v, sem1, sem2):
    wid = lax.axis_index("s") * NC + lax.axis_index("c")
    base = wid * NPW

    @pl.loop(0, NCHUNK)
    def _chunk(ci):
        nb = base + ci * CH
        rb = nb * MAX_NB
        pltpu.sync_copy(ia_hbm.at[pl.ds(rb, ROWS)], ia_v)
        pltpu.sync_copy(ib_hbm.at[pl.ds(rb, ROWS)], ib_v)
        cpa = pltpu.async_copy(t1_hbm.at[ia_v], r1_v, sem1)
        cpb = pltpu.async_copy(t2_hbm.at[ib_v], r2_v, sem2)
        cpa.wait()
        cpb.wait()

        @pl.loop(0, NCB)
        def _cb(cb):
            c = cb * L
            for n in range(CH):
                acc = jnp.zeros((L,), jnp.float32)
                for k in range(MAX_NB):
                    x1 = r1_v[n * MAX_NB + k, pl.ds(c, L)]
                    x2 = r2_v[n * MAX_NB + k, pl.ds(c, L)]
                    if mode == "relu":
                        acc = acc + jnp.maximum(x1 + x2, 0.0)
                    else:
                        acc = acc + x1 * x2
                o_v[n, pl.ds(c, L)] = acc

        pltpu.sync_copy(o_v, o_hbm.at[pl.ds(nb, CH)])


def _make_sc_stage(mode):
    mesh = plsc.VectorSubcoreMesh(core_axis_name="c", subcore_axis_name="s")
    return pl.kernel(
        functools.partial(_sc_stage_body, mode),
        out_type=jax.ShapeDtypeStruct((BN, D), jnp.float32),
        mesh=mesh,
        compiler_params=pltpu.CompilerParams(use_tc_tiling_on_sc=False),
        scratch_types=[
            pltpu.VMEM((ROWS,), jnp.int32),
            pltpu.VMEM((ROWS,), jnp.int32),
            pltpu.VMEM((ROWS, D), jnp.float32),
            pltpu.VMEM((ROWS, D), jnp.float32),
            pltpu.VMEM((CH, D), jnp.float32),
            pltpu.SemaphoreType.DMA,
            pltpu.SemaphoreType.DMA,
        ],
    )


_sc_relu = _make_sc_stage("relu")
_sc_prod = _make_sc_stage("prod")


def kernel(input_atom, input_bond, atom_graph, bond_graph, num_nbs, node_mask,
           placeholder1, placeholder2,
           W_atom, W_nei_atom, W_nei_bond, W_self, W_U2, b_U2, W_U1, b_U1):
    f32 = jnp.float32
    atom = jnp.pad(input_atom.reshape(BN, ATOM_FDIM), ((0, 0), (0, AF_P - ATOM_FDIM)))
    bond = jnp.pad(input_bond.reshape(BN, BOND_FDIM), ((0, 0), (0, BF_P - BOND_FDIM)))

    pad_h = D - HIDDEN
    wa = jnp.pad(W_atom, ((0, AF_P - ATOM_FDIM), (0, pad_h)))
    wnb = jnp.pad(W_nei_bond, ((0, BF_P - BOND_FDIM), (0, pad_h)))
    wu2a = jnp.pad(W_U2[:HIDDEN], ((0, pad_h), (0, pad_h)))
    wu2b = jnp.pad(W_U2[HIDDEN:], ((0, BF_P - BOND_FDIM), (0, pad_h)))
    bu2 = jnp.pad(b_U2, (0, pad_h)).reshape(1, D)
    wu1a = jnp.pad(W_U1[:HIDDEN], ((0, pad_h), (0, pad_h)))
    wu1b = jnp.pad(W_U1[HIDDEN:], ((0, pad_h), (0, pad_h)))
    bu1 = jnp.pad(b_U1, (0, pad_h)).reshape(1, D)
    wna = jnp.pad(W_nei_atom, ((0, pad_h), (0, pad_h)))
    ws = jnp.pad(W_self, ((0, pad_h), (0, pad_h)))

    mask = jnp.arange(MAX_NB, dtype=jnp.int32)[None, None, :] < num_nbs[:, :, None]
    aflat = jnp.where(mask, atom_graph[..., 0] * N + atom_graph[..., 1], BN)
    bflat = jnp.where(mask, bond_graph[..., 0] * N + bond_graph[..., 1], BN)
    aflat = aflat.reshape(BN * MAX_NB).astype(jnp.int32)
    bflat = bflat.reshape(BN * MAX_NB).astype(jnp.int32)

    af0, q0, hbz, fbz = _run_tc(
        _tc_prep, [(BN, D), (DT, D), (DT, D), (DT, D)],
        atom, bond, wa, wu2a, wnb, wu2b, bu2)

    nl0 = _sc_relu(q0, fbz, aflat, bflat)
    af1, q1 = _run_tc(_tc_mid, [(BN, D), (DT, D)],
                      af0, nl0, wu1a, wu1b, bu1, wu2a)
    nl1 = _sc_relu(q1, fbz, aflat, bflat)
    pz, s2 = _run_tc(_tc_last, [(DT, D), (BN, D)],
                     af1, nl1, wu1a, wu1b, bu1, wna, ws)
    fn = _sc_prod(pz, hbz, aflat, bflat)

    nm = node_mask.reshape(BN, 1).astype(f32)
    (out,) = _run_tc(_tc_out, [(BN, D)], s2, fn, nm)
    return out[:, :HIDDEN].reshape(B, N, HIDDEN)
